# baseline (speedup 1.0000x reference)
import numpy as np

import jax
import jax.numpy as jnp
from jax import lax
from jax.experimental import pallas as pl
from jax.experimental.pallas import tpu as pltpu
from jax.experimental.pallas import tpu_sc as plsc

NC = 2
NS = 16
NW = NC * NS
L = 16

B = 4096
D = 64
HIST = 50
WISH = 20
HW = 80
B_PER_W = B // NW
DEPTH = 8

PERM = np.concatenate([np.r_[0:32:2], np.r_[1:32:2],
                       np.r_[32:64:2], np.r_[33:64:2]])


def _sc_pool_kernel(tbl_hbm, hw_idx_hbm, u0_hbm, item_hbm,
                    hw_idx_v, book_buf, u0_st, it_st,
                    sem0, sem1, sem2, sem3, sem4, sem5, sem6, sem7):
    sems = (sem0, sem1, sem2, sem3, sem4, sem5, sem6, sem7)
    wid = lax.axis_index("s") * NC + lax.axis_index("c")
    base = wid * B_PER_W

    pltpu.sync_copy(hw_idx_hbm.at[pl.ds(base, B_PER_W)], hw_idx_v)

    def issue(r, t):
        pltpu.async_copy(tbl_hbm.at[hw_idx_v.at[r]], book_buf.at[t], sems[t])

    def wait_slot(r, t):
        pltpu.make_async_copy(tbl_hbm.at[hw_idx_v.at[r]], book_buf.at[t],
                              sems[t]).wait()

    for r0 in range(DEPTH - 1):
        issue(r0, r0)

    zero = jnp.zeros((L,), jnp.float32)
    unpack = lambda v: plsc.unpack(v, format=plsc.PackFormat.INTERLEAVED)

    def acc_rows(t, j0, j1):
        acc = [zero] * 4
        for j in range(j0, j1):
            for h in range(2):
                a, b = unpack(book_buf[t, j, pl.ds(32 * h, 32)])
                acc[2 * h] = acc[2 * h] + a
                acc[2 * h + 1] = acc[2 * h + 1] + b
        return acc

    def accum(r, t):
        uh = acc_rows(t, 0, HIST)
        uw = acc_rows(t, HIST, HIST + WISH)
        bal = acc_rows(t, 70, 73)
        tg = acc_rows(t, 73, 78)
        for c in range(4):
            sl = pl.ds(c * L, L)
            u0_st[r, sl] = uh[c] * (1.0 / 50.0) + uw[c] * (1.0 / 20.0)
            it_st[r, sl] = bal[c] + tg[c] * (1.0 / 5.0)

    def body(i, carry):
        for s in range(DEPTH):
            r = i * DEPTH + s
            wait_slot(r, s)
            accum(r, s)
            nxt = r + DEPTH - 1

            @pl.when(nxt < B_PER_W)
            def _():
                issue(nxt, (s + DEPTH - 1) % DEPTH)
        return carry

    lax.fori_loop(0, B_PER_W // DEPTH, body, 0)

    pltpu.sync_copy(u0_st, u0_hbm.at[pl.ds(base, B_PER_W)])
    pltpu.sync_copy(it_st, item_hbm.at[pl.ds(base, B_PER_W)])


def _sc_pool(table, hw_idx):
    mesh = plsc.VectorSubcoreMesh(core_axis_name="c", subcore_axis_name="s")
    f32 = jnp.float32
    kern = pl.kernel(
        _sc_pool_kernel,
        out_type=(jax.ShapeDtypeStruct((B, D), f32),
                  jax.ShapeDtypeStruct((B, D), f32)),
        mesh=mesh,
        compiler_params=pltpu.CompilerParams(use_tc_tiling_on_sc=False,
                                             needs_layout_passes=False),
        scratch_types=(
            pltpu.VMEM((B_PER_W, HW), jnp.int32),
            pltpu.VMEM((DEPTH, HW, D), jnp.bfloat16),
            pltpu.VMEM((B_PER_W, D), f32),
            pltpu.VMEM((B_PER_W, D), f32),
            pltpu.SemaphoreType.DMA,
            pltpu.SemaphoreType.DMA,
            pltpu.SemaphoreType.DMA,
            pltpu.SemaphoreType.DMA,
            pltpu.SemaphoreType.DMA,
            pltpu.SemaphoreType.DMA,
            pltpu.SemaphoreType.DMA,
            pltpu.SemaphoreType.DMA,
        ),
    )
    return kern(table, hw_idx)


def _tc_mlp_kernel(u0_ref, item_ref, dense_ref,
                   dw1_ref, db1_ref, dw2_ref, db2_ref,
                   uw1_ref, ub1_ref, uw2_ref, ub2_ref, uw3_ref, ub3_ref,
                   out_ref):
    f32 = jnp.float32
    u0 = u0_ref[...]
    h = jax.nn.relu(jnp.dot(u0, uw1_ref[...], preferred_element_type=f32)
                    + ub1_ref[...])
    h = jax.nn.relu(jnp.dot(h, uw2_ref[...], preferred_element_type=f32)
                    + ub2_ref[...])
    u_emb = jnp.dot(h, uw3_ref[...], preferred_element_type=f32) + ub3_ref[...]
    d = jax.nn.relu(jnp.dot(dense_ref[...], dw1_ref[...],
                            preferred_element_type=f32) + db1_ref[...])
    d_e = jnp.dot(d, dw2_ref[...], preferred_element_type=f32) + db2_ref[...]
    i_emb = item_ref[...] + d_e
    out_ref[...] = jnp.sum(u_emb * i_emb, axis=1, keepdims=True)


def _tc_mlp(u0, item_pool, dense8,
            dW1, db1, dW2, db2, uW1, ub1, uW2, ub2, uW3, ub3):
    f32 = jnp.float32
    BLK = 512
    grid = (B // BLK,)

    def batch_spec(cols):
        return pl.BlockSpec((BLK, cols), lambda i: (i, 0))

    def full_spec(a):
        return pl.BlockSpec(a.shape, lambda i: (0,) * a.ndim)

    return pl.pallas_call(
        _tc_mlp_kernel,
        grid=grid,
        in_specs=[
            batch_spec(D), batch_spec(D), batch_spec(8),
            full_spec(dW1), full_spec(db1), full_spec(dW2), full_spec(db2),
            full_spec(uW1), full_spec(ub1), full_spec(uW2), full_spec(ub2),
            full_spec(uW3), full_spec(ub3),
        ],
        out_specs=pl.BlockSpec((BLK, 1), lambda i: (i, 0)),
        out_shape=jax.ShapeDtypeStruct((B, 1), f32),
    )(u0, item_pool, dense8,
      dW1, db1, dW2, db2, uW1, ub1, uW2, ub2, uW3, ub3)


def kernel(hist_ids, wish_ids, bid, auth, lang, tags, dense,
           book_emb, auth_emb, lang_emb, tag_emb,
           dense_W1, dense_b1, dense_W2, dense_b2,
           user_W1, user_b1, user_W2, user_b2, user_W3, user_b3):
    i32 = jnp.int32
    f32 = jnp.float32
    bf16 = jnp.bfloat16

    n_book = book_emb.shape[0]
    n_tag = tag_emb.shape[0]
    n_auth = auth_emb.shape[0]
    table = jnp.concatenate([book_emb, tag_emb, auth_emb, lang_emb],
                            axis=0).astype(bf16)
    tag_off = n_book
    auth_off = n_book + n_tag
    lang_off = n_book + n_tag + n_auth

    zcol = jnp.zeros((B, 2), i32)
    hw_idx = jnp.concatenate(
        [hist_ids.astype(i32), wish_ids.astype(i32),
         bid.astype(i32).reshape(B, 1),
         auth.astype(i32).reshape(B, 1) + auth_off,
         lang.astype(i32).reshape(B, 1) + lang_off,
         tags.astype(i32) + tag_off, zcol], axis=1)

    u0, item_pool = _sc_pool(table, hw_idx)

    perm = jnp.asarray(PERM)
    dense8 = jnp.pad(dense.astype(f32), ((0, 0), (0, 8 - dense.shape[1])))
    dW1 = jnp.pad(dense_W1, ((0, 8 - dense_W1.shape[0]), (0, 0)))
    out = _tc_mlp(u0, item_pool, dense8,
                  dW1, dense_b1.reshape(1, -1),
                  dense_W2[:, perm], dense_b2[perm].reshape(1, -1),
                  user_W1[perm, :], user_b1.reshape(1, -1),
                  user_W2, user_b2.reshape(1, -1),
                  user_W3[:, perm], user_b3[perm].reshape(1, -1))
    return out
